# Initial kernel scaffold; baseline (speedup 1.0000x reference)
import jax
import jax.numpy as jnp
from jax.experimental import pallas as pl


def kernel(x_nchw, w_oihw, bias, gamma, beta):
    raise NotImplementedError("write your pallas kernel here")


# speedup vs baseline: 1.7443x; 1.7443x over previous
import functools

import jax
import jax.numpy as jnp
from jax import lax
from jax.experimental import pallas as pl
from jax.experimental.pallas import tpu as pltpu


def _shift_lanes(x, s):
    if s == 0:
        return x
    C, L = x.shape
    zeros = jnp.zeros((C, abs(s)), x.dtype)
    if s > 0:
        return jnp.concatenate([x[:, s:], zeros], axis=1)
    return jnp.concatenate([zeros, x[:, : L + s]], axis=1)


def _assemble_patches(x_f32, W):
    Cin, HW = x_f32.shape
    xb = x_f32.astype(jnp.bfloat16)
    col = lax.broadcasted_iota(jnp.int32, (Cin, HW), 1)
    if W & (W - 1) == 0:
        col = col & (W - 1)
    else:
        col = lax.rem(col, W)
    xm = jnp.where(col == 0, jnp.bfloat16(0), _shift_lanes(xb, -1))
    xp = jnp.where(col == W - 1, jnp.bfloat16(0), _shift_lanes(xb, 1))
    rows = []
    for dy in (-1, 0, 1):
        for xv in (xm, xb, xp):
            rows.append(_shift_lanes(xv, dy * W))
    return jnp.concatenate(rows, axis=0)


def _stats_kernel(x_ref, w_ref, stat_ref, *, W):
    patches = _assemble_patches(x_ref[...], W)
    acc = jnp.dot(w_ref[...], patches,
                  preferred_element_type=jnp.float32)
    s1 = jnp.sum(acc, axis=1, keepdims=True)
    s2 = jnp.sum(acc * acc, axis=1, keepdims=True)
    stat_ref[...] = jnp.concatenate([s1, s2], axis=1)


def _fused_kernel(x_ref, w_ref, scale_ref, shift_ref, out_ref, *, W):
    patches = _assemble_patches(x_ref[...], W)
    acc = jnp.dot(w_ref[...], patches,
                  preferred_element_type=jnp.float32)
    out_ref[...] = jnp.tanh(acc * scale_ref[...] + shift_ref[...])


@jax.jit
def _conv_bn_tanh(x_nchw, w_oihw, gamma, beta, eps=1e-5):
    N, Cin, H, W = x_nchw.shape
    Cout = w_oihw.shape[0]
    HW = H * W
    K = 9 * Cin
    f32 = jnp.float32

    x_flat = x_nchw.reshape(N, Cin, HW)
    w_mat = jnp.transpose(w_oihw, (0, 2, 3, 1)).reshape(Cout, K)
    w_mat = w_mat.astype(jnp.bfloat16)

    x_spec = pl.BlockSpec((None, Cin, HW), lambda n: (n, 0, 0))
    w_spec = pl.BlockSpec((Cout, K), lambda n: (0, 0))
    vmem_limit = 100 * 1024 * 1024

    matmul_flops = 2 * N * HW * K * Cout

    stats = pl.pallas_call(
        functools.partial(_stats_kernel, W=W),
        out_shape=jax.ShapeDtypeStruct((N, Cout, 2), f32),
        grid=(N,),
        in_specs=[x_spec, w_spec],
        out_specs=pl.BlockSpec((None, Cout, 2), lambda n: (n, 0, 0)),
        compiler_params=pltpu.CompilerParams(
            dimension_semantics=("parallel",),
            vmem_limit_bytes=vmem_limit),
        cost_estimate=pl.CostEstimate(
            flops=matmul_flops, transcendentals=0,
            bytes_accessed=x_flat.size * 4 + N * Cout * 2 * 4),
    )(x_flat, w_mat)

    count = jnp.asarray(N * HW, f32)
    ssum = jnp.sum(stats[:, :, 0], axis=0)
    ssq = jnp.sum(stats[:, :, 1], axis=0)
    mean = ssum / count
    var = jnp.maximum(ssq / count - mean * mean, 0.0)
    invstd = lax.rsqrt(var + jnp.asarray(eps, f32))
    g = gamma.astype(f32) * invstd
    scale = g.reshape(Cout, 1)
    shift = (beta.astype(f32) - mean * g).reshape(Cout, 1)

    y_flat = pl.pallas_call(
        functools.partial(_fused_kernel, W=W),
        out_shape=jax.ShapeDtypeStruct((N, Cout, HW), x_nchw.dtype),
        grid=(N,),
        in_specs=[x_spec, w_spec,
                  pl.BlockSpec((Cout, 1), lambda n: (0, 0)),
                  pl.BlockSpec((Cout, 1), lambda n: (0, 0))],
        out_specs=pl.BlockSpec((None, Cout, HW), lambda n: (n, 0, 0)),
        compiler_params=pltpu.CompilerParams(
            dimension_semantics=("parallel",),
            vmem_limit_bytes=vmem_limit),
        cost_estimate=pl.CostEstimate(
            flops=matmul_flops + 3 * N * HW * Cout,
            transcendentals=N * HW * Cout,
            bytes_accessed=x_flat.size * 4 + N * Cout * HW * 4),
    )(x_flat, w_mat, scale, shift)

    return y_flat.reshape(N, Cout, H, W)


def kernel(x_nchw, w_oihw, bias, gamma, beta):
    del bias
    return _conv_bn_tanh(x_nchw, w_oihw, gamma, beta)


# speedup vs baseline: 1.8376x; 1.0535x over previous
import functools

import jax
import jax.numpy as jnp
from jax import lax
from jax.experimental import pallas as pl
from jax.experimental.pallas import tpu as pltpu


def _shift_lanes(x, s):
    if s == 0:
        return x
    C, L = x.shape
    zeros = jnp.zeros((C, abs(s)), x.dtype)
    if s > 0:
        return jnp.concatenate([x[:, s:], zeros], axis=1)
    return jnp.concatenate([zeros, x[:, : L + s]], axis=1)


def _assemble_patches(x_f32, W):
    Cin, HW = x_f32.shape
    xb = x_f32.astype(jnp.bfloat16)
    col = lax.broadcasted_iota(jnp.int32, (Cin, HW), 1)
    if W & (W - 1) == 0:
        col = col & (W - 1)
    else:
        col = lax.rem(col, W)
    xm = jnp.where(col == 0, jnp.bfloat16(0), _shift_lanes(xb, -1))
    xp = jnp.where(col == W - 1, jnp.bfloat16(0), _shift_lanes(xb, 1))
    rows = []
    for dy in (-1, 0, 1):
        for xv in (xm, xb, xp):
            rows.append(_shift_lanes(xv, dy * W))
    return jnp.concatenate(rows, axis=0)


def _stats_kernel(x_ref, w_ref, stat_ref, *, W, NB):
    for b in range(NB):
        patches = _assemble_patches(x_ref[b], W)
        acc = jnp.dot(w_ref[...], patches,
                      preferred_element_type=jnp.float32)
        s1 = jnp.sum(acc, axis=1, keepdims=True)
        s2 = jnp.sum(acc * acc, axis=1, keepdims=True)
        stat_ref[b] = jnp.concatenate([s1, s2], axis=1)


def _fused_kernel(x_ref, w_ref, scale_ref, shift_ref, out_ref, *, W, NB):
    for b in range(NB):
        patches = _assemble_patches(x_ref[b], W)
        acc = jnp.dot(w_ref[...], patches,
                      preferred_element_type=jnp.float32)
        out_ref[b] = jnp.tanh(acc * scale_ref[...] + shift_ref[...])


@jax.jit
def _conv_bn_tanh(x_nchw, w_oihw, gamma, beta, eps=1e-5):
    N, Cin, H, W = x_nchw.shape
    Cout = w_oihw.shape[0]
    HW = H * W
    K = 9 * Cin
    f32 = jnp.float32

    NB = 4 if N % 4 == 0 else 1
    x_flat = x_nchw.reshape(N, Cin, HW)
    w_mat = jnp.transpose(w_oihw, (0, 2, 3, 1)).reshape(Cout, K)
    w_mat = w_mat.astype(jnp.bfloat16)

    x_spec = pl.BlockSpec((NB, Cin, HW), lambda n: (n, 0, 0))
    w_spec = pl.BlockSpec((Cout, K), lambda n: (0, 0))
    vmem_limit = 100 * 1024 * 1024

    matmul_flops = 2 * N * HW * K * Cout

    stats = pl.pallas_call(
        functools.partial(_stats_kernel, W=W, NB=NB),
        out_shape=jax.ShapeDtypeStruct((N, Cout, 2), f32),
        grid=(N // NB,),
        in_specs=[x_spec, w_spec],
        out_specs=pl.BlockSpec((NB, Cout, 2), lambda n: (n, 0, 0)),
        compiler_params=pltpu.CompilerParams(
            dimension_semantics=("parallel",),
            vmem_limit_bytes=vmem_limit),
        cost_estimate=pl.CostEstimate(
            flops=matmul_flops, transcendentals=0,
            bytes_accessed=x_flat.size * 4 + N * Cout * 2 * 4),
    )(x_flat, w_mat)

    count = jnp.asarray(N * HW, f32)
    ssum = jnp.sum(stats[:, :, 0], axis=0)
    ssq = jnp.sum(stats[:, :, 1], axis=0)
    mean = ssum / count
    var = jnp.maximum(ssq / count - mean * mean, 0.0)
    invstd = lax.rsqrt(var + jnp.asarray(eps, f32))
    g = gamma.astype(f32) * invstd
    scale = g.reshape(Cout, 1)
    shift = (beta.astype(f32) - mean * g).reshape(Cout, 1)

    y_flat = pl.pallas_call(
        functools.partial(_fused_kernel, W=W, NB=NB),
        out_shape=jax.ShapeDtypeStruct((N, Cout, HW), x_nchw.dtype),
        grid=(N // NB,),
        in_specs=[x_spec, w_spec,
                  pl.BlockSpec((Cout, 1), lambda n: (0, 0)),
                  pl.BlockSpec((Cout, 1), lambda n: (0, 0))],
        out_specs=pl.BlockSpec((NB, Cout, HW), lambda n: (n, 0, 0)),
        compiler_params=pltpu.CompilerParams(
            dimension_semantics=("parallel",),
            vmem_limit_bytes=vmem_limit),
        cost_estimate=pl.CostEstimate(
            flops=matmul_flops + 3 * N * HW * Cout,
            transcendentals=N * HW * Cout,
            bytes_accessed=x_flat.size * 4 + N * Cout * HW * 4),
    )(x_flat, w_mat, scale, shift)

    return y_flat.reshape(N, Cout, H, W)


def kernel(x_nchw, w_oihw, bias, gamma, beta):
    del bias
    return _conv_bn_tanh(x_nchw, w_oihw, gamma, beta)
